# baseline (speedup 1.0000x reference)
import jax
import jax.numpy as jnp
from jax.experimental import pallas as pl
from jax.experimental.pallas import tpu as pltpu

BLOCK = 10000


def _encode_block(a_ref, b_ref, w_ref, o_ref):
    d = a_ref.shape[1]
    acc = jnp.dot(a_ref[:], w_ref[0:d, :], preferred_element_type=jnp.float32)
    acc = acc + jnp.dot(b_ref[:], w_ref[d:, :], preferred_element_type=jnp.float32)
    o_ref[:] = jnp.maximum(acc, 0.0)


def kernel(edges_attr, edges_nb_attr, W):
    e, d = edges_attr.shape
    n_out = W.shape[1]
    block = min(e, BLOCK)
    while e % block:
        block -= 1
    return pl.pallas_call(
        _encode_block,
        grid=(e // block,),
        in_specs=[
            pl.BlockSpec((block, d), lambda i: (i, 0)),
            pl.BlockSpec((block, d), lambda i: (i, 0)),
            pl.BlockSpec((2 * d, n_out), lambda i: (0, 0)),
        ],
        out_specs=pl.BlockSpec((block, n_out), lambda i: (i, 0)),
        out_shape=jax.ShapeDtypeStruct((e, n_out), jnp.float32),
        compiler_params=pltpu.CompilerParams(
            dimension_semantics=("parallel",),
        ),
    )(edges_attr, edges_nb_attr, W)
